# baseline (speedup 1.0000x reference)
import functools

import jax
import jax.numpy as jnp
from jax import lax
from jax.experimental import pallas as pl
from jax.experimental.pallas import tpu as pltpu
from jax.experimental.pallas import tpu_sc as plsc

NUM_EMBEDDINGS = 1000000
EMBED_DIM = 32
BATCH = 1024
TOKEN_LEN = 200
B = BATCH * TOKEN_LEN

_info = plsc.get_sparse_core_info()
NC, NS, L = _info.num_cores, _info.num_subcores, _info.num_lanes
NW = NC * NS


def _make_gather():
    mesh = plsc.VectorSubcoreMesh(core_axis_name="c", subcore_axis_name="s")

    @functools.partial(
        pl.kernel,
        mesh=mesh,
        out_type=jax.ShapeDtypeStruct((TOKEN_LEN, 4, 8, 8, 128), jnp.float32),
        compiler_params=pltpu.CompilerParams(
            use_tc_tiling_on_sc=False, needs_layout_passes=False
        ),
        scratch_types=[
            pltpu.VMEM((BATCH,), jnp.int32),
            pltpu.VMEM((BATCH, EMBED_DIM), jnp.float32),
            pltpu.VMEM((4, 8, 8, 128), jnp.float32),
            pltpu.SemaphoreType.DMA,
        ],
    )
    def k(idx_hbm, table_hbm, out_hbm, idx_v, rows_v, tout_v, sem):
        wid = lax.axis_index("s") * NC + lax.axis_index("c")
        nt = jnp.where(wid < 8, 7, 6)
        t_base = jnp.where(wid < 8, wid * 7, 56 + (wid - 8) * 6)

        def per_token(lt, _):
            t = t_base + lt
            i = t // 8
            r = t % 8
            for j in range(8):
                pltpu.sync_copy(
                    idx_hbm.at[pl.ds(i * 8192 + j * 1024 + r * 128, 128)],
                    idx_v.at[pl.ds(j * 128, 128)],
                )
            pltpu.async_copy(table_hbm.at[idx_v], rows_v, sem).wait()

            def per_chunk(cc, _):
                jb = cc // 8
                c0l = (cc % 8) * L
                c_vec = lax.iota(jnp.int32, L) + cc * L
                for f in range(EMBED_DIM):
                    f_vec = jnp.full((L,), f, jnp.int32)
                    v = plsc.load_gather(rows_v, [c_vec, f_vec])
                    tout_v[f // 8, jb, f % 8, pl.ds(c0l, L)] = v
                return 0

            lax.fori_loop(0, BATCH // L, per_chunk, 0, unroll=False)
            pltpu.sync_copy(tout_v, out_hbm.at[t])
            return 0

        lax.fori_loop(0, nt, per_token, 0, unroll=False)

    return k


_gather = _make_gather()


@jax.jit
def kernel(x, table):
    xt = x.T.reshape(25, 8, 8, 128).transpose(0, 2, 1, 3).reshape(B)
    out = _gather(xt, table)
    return out.transpose(2, 4, 0, 1, 3).reshape(BATCH, TOKEN_LEN, EMBED_DIM)


# speedup vs baseline: 1.1408x; 1.1408x over previous
import functools

import jax
import jax.numpy as jnp
from jax import lax
from jax.experimental import pallas as pl
from jax.experimental.pallas import tpu as pltpu
from jax.experimental.pallas import tpu_sc as plsc

NUM_EMBEDDINGS = 1000000
EMBED_DIM = 32
BATCH = 1024
TOKEN_LEN = 200
B = BATCH * TOKEN_LEN

_info = plsc.get_sparse_core_info()
NC, NS, L = _info.num_cores, _info.num_subcores, _info.num_lanes
NW = NC * NS

_TC_GRID = 489
_SEG = _TC_GRID * 512
_RM_ROWS = 4 * _SEG


def _tc_retile_body(t0, t1, t2, t3, o_ref):
    o_ref[...] = jnp.concatenate(
        [t0[...].T, t1[...].T, t2[...].T, t3[...].T], axis=1
    )


_tc_retile = pl.pallas_call(
    _tc_retile_body,
    grid=(_TC_GRID,),
    in_specs=[
        pl.BlockSpec(
            (EMBED_DIM, 512),
            lambda g, s=s: (0, jnp.minimum(_TC_GRID * s + g, NUM_EMBEDDINGS // 512)),
        )
        for s in range(4)
    ],
    out_specs=pl.BlockSpec((512, 128), lambda g: (g, 0)),
    out_shape=jax.ShapeDtypeStruct((_SEG, 128), jnp.float32),
)


def _make_gather():
    mesh = plsc.VectorSubcoreMesh(core_axis_name="c", subcore_axis_name="s")

    @functools.partial(
        pl.kernel,
        mesh=mesh,
        out_type=jax.ShapeDtypeStruct((TOKEN_LEN, 4, 8, 8, 128), jnp.float32),
        compiler_params=pltpu.CompilerParams(
            use_tc_tiling_on_sc=False, needs_layout_passes=False
        ),
        scratch_types=[
            pltpu.VMEM((7 * BATCH,), jnp.int32),
            pltpu.VMEM((BATCH, EMBED_DIM), jnp.float32),
            pltpu.VMEM((BATCH, EMBED_DIM), jnp.float32),
            pltpu.VMEM((4, 8, 8, 128), jnp.float32),
            pltpu.SemaphoreType.DMA,
            pltpu.SemaphoreType.DMA,
        ],
    )
    def k(idx_hbm, table_hbm, out_hbm, idx_v, rows0, rows1, tout_v, gs0, gs1):
        wid = lax.axis_index("s") * NC + lax.axis_index("c")
        nt = jnp.where(wid < 8, 7, 6)
        t_base = jnp.where(wid < 8, wid * 7, 56 + (wid - 8) * 6)

        pltpu.sync_copy(
            idx_hbm.at[pl.ds(t_base * BATCH, 6 * BATCH)],
            idx_v.at[pl.ds(0, 6 * BATCH)],
        )

        @pl.when(nt == 7)
        def _():
            pltpu.sync_copy(
                idx_hbm.at[pl.ds((t_base + 6) * BATCH, BATCH)],
                idx_v.at[pl.ds(6 * BATCH, BATCH)],
            )

        def remap(ci, _):
            v = idx_v[pl.ds(ci * L, L)]
            s = v // _SEG
            idx_v[pl.ds(ci * L, L)] = v * 4 - s * (4 * _SEG - 1)
            return 0

        lax.fori_loop(0, 7 * BATCH // L, remap, 0, unroll=False)

        rows = (rows0, rows1)
        gsem = (gs0, gs1)

        def gather_start(lt):
            return pltpu.async_copy(
                table_hbm.at[idx_v.at[pl.ds(lt * BATCH, BATCH)]],
                rows[lt % 2],
                gsem[lt % 2],
            )

        g = [gather_start(0), None]

        for lt in range(7):

            @pl.when(lt < nt)
            def _(lt=lt):
                if lt + 1 < 7:

                    @pl.when(lt + 1 < nt)
                    def _():
                        g[(lt + 1) % 2] = gather_start(lt + 1)

                g[lt % 2].wait()
                src = rows[lt % 2]

                def per_chunk(cc, _):
                    jb = cc // 8
                    c0l = (cc % 8) * L
                    c_vec = lax.iota(jnp.int32, L) + cc * L
                    for f in range(EMBED_DIM):
                        f_vec = jnp.full((L,), f, jnp.int32)
                        v = plsc.load_gather(src, [c_vec, f_vec])
                        tout_v[f // 8, jb, f % 8, pl.ds(c0l, L)] = v
                    return 0

                lax.fori_loop(0, BATCH // L, per_chunk, 0, unroll=False)
                pltpu.sync_copy(tout_v, out_hbm.at[t_base + lt])

    return k


_gather = _make_gather()


@jax.jit
def kernel(x, table):
    tt = table.T
    table_rm = _tc_retile(tt, tt, tt, tt).reshape(_RM_ROWS, EMBED_DIM)
    out = _gather(x.T.reshape(B), table_rm)
    return out.transpose(2, 4, 0, 1, 3).reshape(BATCH, TOKEN_LEN, EMBED_DIM)


# speedup vs baseline: 1.3219x; 1.1588x over previous
import functools

import jax
import jax.numpy as jnp
from jax import lax
from jax.experimental import pallas as pl
from jax.experimental.pallas import tpu as pltpu
from jax.experimental.pallas import tpu_sc as plsc

NUM_EMBEDDINGS = 1000000
EMBED_DIM = 32
BATCH = 1024
TOKEN_LEN = 200
B = BATCH * TOKEN_LEN

_info = plsc.get_sparse_core_info()
NC, NS, L = _info.num_cores, _info.num_subcores, _info.num_lanes
NW = NC * NS

_TC_GRID = 489
_SEG = _TC_GRID * 512
_RM_ROWS = 4 * _SEG


def _tc_retile_body(t0, t1, t2, t3, o_ref):
    o_ref[...] = jnp.concatenate(
        [t0[...].T, t1[...].T, t2[...].T, t3[...].T], axis=1
    )


_tc_retile = pl.pallas_call(
    _tc_retile_body,
    grid=(_TC_GRID,),
    in_specs=[
        pl.BlockSpec(
            (EMBED_DIM, 512),
            lambda g, s=s: (0, jnp.minimum(_TC_GRID * s + g, NUM_EMBEDDINGS // 512)),
        )
        for s in range(4)
    ],
    out_specs=pl.BlockSpec((512, 128), lambda g: (g, 0)),
    out_shape=jax.ShapeDtypeStruct((_SEG, 128), jnp.float32),
)


def _make_gather():
    mesh = plsc.VectorSubcoreMesh(core_axis_name="c", subcore_axis_name="s")

    @functools.partial(
        pl.kernel,
        mesh=mesh,
        out_type=jax.ShapeDtypeStruct((TOKEN_LEN, 4, 8, 8, 128), jnp.float32),
        compiler_params=pltpu.CompilerParams(
            use_tc_tiling_on_sc=False, needs_layout_passes=False
        ),
        scratch_types=[
            pltpu.VMEM((7 * BATCH,), jnp.int32),
            pltpu.VMEM((BATCH, EMBED_DIM), jnp.float32),
            pltpu.VMEM((BATCH, EMBED_DIM), jnp.float32),
            pltpu.VMEM((4, 8, 8, 128), jnp.float32),
            pltpu.SemaphoreType.DMA,
            pltpu.SemaphoreType.DMA,
        ],
    )
    def k(idx_hbm, table_hbm, out_hbm, idx_v, rows0, rows1, tout_v, gs0, gs1):
        wid = lax.axis_index("s") * NC + lax.axis_index("c")
        nt = jnp.where(wid < 8, 7, 6)
        t_base = jnp.where(wid < 8, wid * 7, 56 + (wid - 8) * 6)

        pltpu.sync_copy(
            idx_hbm.at[pl.ds(t_base * BATCH, 6 * BATCH)],
            idx_v.at[pl.ds(0, 6 * BATCH)],
        )

        @pl.when(nt == 7)
        def _():
            pltpu.sync_copy(
                idx_hbm.at[pl.ds((t_base + 6) * BATCH, BATCH)],
                idx_v.at[pl.ds(6 * BATCH, BATCH)],
            )

        def remap(ci, _):
            v = idx_v[pl.ds(ci * L, L)]
            s = v // _SEG
            idx_v[pl.ds(ci * L, L)] = v * 4 - s * (4 * _SEG - 1)
            return 0

        lax.fori_loop(0, 7 * BATCH // L, remap, 0, unroll=False)

        rows = (rows0, rows1)
        gsem = (gs0, gs1)

        def gather_start(lt):
            return pltpu.async_copy(
                table_hbm.at[idx_v.at[pl.ds(lt * BATCH, BATCH)]],
                rows[lt % 2],
                gsem[lt % 2],
            )

        g = [gather_start(0), None]

        for lt in range(7):

            @pl.when(lt < nt)
            def _(lt=lt):
                if lt + 1 < 7:

                    @pl.when(lt + 1 < nt)
                    def _():
                        g[(lt + 1) % 2] = gather_start(lt + 1)

                g[lt % 2].wait()
                src = rows[lt % 2]

                def per_chunk(cc, _):
                    jb = cc // 8
                    c0l = (cc % 8) * L
                    c_vec = lax.iota(jnp.int32, L) + cc * L
                    for f in range(EMBED_DIM):
                        f_vec = jnp.full((L,), f, jnp.int32)
                        v = plsc.load_gather(src, [c_vec, f_vec])
                        tout_v[f // 8, jb, f % 8, pl.ds(c0l, L)] = v
                    return 0

                lax.fori_loop(0, BATCH // L, per_chunk, 0, unroll=False)
                pltpu.sync_copy(tout_v, out_hbm.at[t_base + lt])

    return k


_gather = _make_gather()


B_PER_W = B // NW
CHUNK = 1600
NCHUNK = B_PER_W // CHUNK


def _make_simple_gather():
    mesh = plsc.VectorSubcoreMesh(core_axis_name="c", subcore_axis_name="s")

    @functools.partial(
        pl.kernel,
        mesh=mesh,
        out_type=jax.ShapeDtypeStruct((B, EMBED_DIM), jnp.float32),
        compiler_params=pltpu.CompilerParams(use_tc_tiling_on_sc=False),
        scratch_types=[
            pltpu.VMEM((B_PER_W,), jnp.int32),
            pltpu.VMEM((CHUNK, EMBED_DIM), jnp.float32),
            pltpu.VMEM((CHUNK, EMBED_DIM), jnp.float32),
            pltpu.SemaphoreType.DMA,
            pltpu.SemaphoreType.DMA,
            pltpu.SemaphoreType.DMA,
            pltpu.SemaphoreType.DMA,
        ],
    )
    def k(idx_hbm, table_hbm, out_hbm, idx_v, rows0, rows1, gs0, gs1, ws0, ws1):
        wid = lax.axis_index("s") * NC + lax.axis_index("c")
        base = wid * B_PER_W
        pltpu.sync_copy(idx_hbm.at[pl.ds(base, B_PER_W)], idx_v)

        rows = (rows0, rows1)
        gsem = (gs0, gs1)
        wsem = (ws0, ws1)

        def gather_start(i):
            return pltpu.async_copy(
                table_hbm.at[idx_v.at[pl.ds(i * CHUNK, CHUNK)]],
                rows[i % 2],
                gsem[i % 2],
            )

        def write_start(i):
            return pltpu.async_copy(
                rows[i % 2],
                out_hbm.at[pl.ds(base + i * CHUNK, CHUNK)],
                wsem[i % 2],
            )

        gathers = [gather_start(0), gather_start(1)]
        writes = [None, None]
        for i in range(NCHUNK):
            b = i % 2
            gathers[b].wait()
            writes[b] = write_start(i)
            if i + 2 < NCHUNK:
                writes[b].wait()
                gathers[b] = gather_start(i + 2)
        writes[0].wait()
        writes[1].wait()

    return k


_simple_gather = _make_simple_gather()


@jax.jit
def kernel(x, table):
    tt = table.T
    table_rm = _tc_retile(tt, tt, tt, tt).reshape(_RM_ROWS, EMBED_DIM)
    xt = x.T.reshape(B)
    seg = xt // _SEG
    idx = xt * 4 - seg * (4 * _SEG - 1)
    out = _simple_gather(idx, table_rm)
    return out.reshape(TOKEN_LEN, BATCH, EMBED_DIM).transpose(1, 0, 2)


# speedup vs baseline: 1.8219x; 1.3782x over previous
import functools

import jax
import jax.numpy as jnp
from jax import lax
from jax.experimental import pallas as pl
from jax.experimental.pallas import tpu as pltpu
from jax.experimental.pallas import tpu_sc as plsc

NUM_EMBEDDINGS = 1000000
EMBED_DIM = 32
BATCH = 1024
TOKEN_LEN = 200
B = BATCH * TOKEN_LEN

_info = plsc.get_sparse_core_info()
NC, NS, L = _info.num_cores, _info.num_subcores, _info.num_lanes
NW = NC * NS

_TC_GRID = 163
_TC_W = 1536
_SEG = _TC_GRID * _TC_W
_RM_ROWS = 4 * _SEG


def _tc_retile_body(t0, t1, t2, t3, o_ref):
    o_ref[...] = jnp.concatenate(
        [t0[...].T, t1[...].T, t2[...].T, t3[...].T], axis=1
    )


_tc_retile = pl.pallas_call(
    _tc_retile_body,
    grid=(_TC_GRID,),
    in_specs=[
        pl.BlockSpec(
            (EMBED_DIM, _TC_W),
            lambda g, s=s: (0, jnp.minimum(_TC_GRID * s + g, NUM_EMBEDDINGS // _TC_W)),
        )
        for s in range(4)
    ],
    out_specs=pl.BlockSpec((_TC_W, 128), lambda g: (g, 0)),
    out_shape=jax.ShapeDtypeStruct((_SEG, 128), jnp.float32),
)


def _make_gather():
    mesh = plsc.VectorSubcoreMesh(core_axis_name="c", subcore_axis_name="s")

    @functools.partial(
        pl.kernel,
        mesh=mesh,
        out_type=jax.ShapeDtypeStruct((TOKEN_LEN, 4, 8, 8, 128), jnp.float32),
        compiler_params=pltpu.CompilerParams(
            use_tc_tiling_on_sc=False, needs_layout_passes=False
        ),
        scratch_types=[
            pltpu.VMEM((7 * BATCH,), jnp.int32),
            pltpu.VMEM((BATCH, EMBED_DIM), jnp.float32),
            pltpu.VMEM((BATCH, EMBED_DIM), jnp.float32),
            pltpu.VMEM((4, 8, 8, 128), jnp.float32),
            pltpu.SemaphoreType.DMA,
            pltpu.SemaphoreType.DMA,
        ],
    )
    def k(idx_hbm, table_hbm, out_hbm, idx_v, rows0, rows1, tout_v, gs0, gs1):
        wid = lax.axis_index("s") * NC + lax.axis_index("c")
        nt = jnp.where(wid < 8, 7, 6)
        t_base = jnp.where(wid < 8, wid * 7, 56 + (wid - 8) * 6)

        pltpu.sync_copy(
            idx_hbm.at[pl.ds(t_base * BATCH, 6 * BATCH)],
            idx_v.at[pl.ds(0, 6 * BATCH)],
        )

        @pl.when(nt == 7)
        def _():
            pltpu.sync_copy(
                idx_hbm.at[pl.ds((t_base + 6) * BATCH, BATCH)],
                idx_v.at[pl.ds(6 * BATCH, BATCH)],
            )

        def remap(ci, _):
            v = idx_v[pl.ds(ci * L, L)]
            s = v // _SEG
            idx_v[pl.ds(ci * L, L)] = v * 4 - s * (4 * _SEG - 1)
            return 0

        lax.fori_loop(0, 7 * BATCH // L, remap, 0, unroll=False)

        rows = (rows0, rows1)
        gsem = (gs0, gs1)

        def gather_start(lt):
            return pltpu.async_copy(
                table_hbm.at[idx_v.at[pl.ds(lt * BATCH, BATCH)]],
                rows[lt % 2],
                gsem[lt % 2],
            )

        g = [gather_start(0), None]

        for lt in range(7):

            @pl.when(lt < nt)
            def _(lt=lt):
                if lt + 1 < 7:

                    @pl.when(lt + 1 < nt)
                    def _():
                        g[(lt + 1) % 2] = gather_start(lt + 1)

                g[lt % 2].wait()
                src = rows[lt % 2]

                def per_chunk(cc, _):
                    jb = cc // 8
                    c0l = (cc % 8) * L
                    c_vec = lax.iota(jnp.int32, L) + cc * L
                    for f in range(EMBED_DIM):
                        f_vec = jnp.full((L,), f, jnp.int32)
                        v = plsc.load_gather(src, [c_vec, f_vec])
                        tout_v[f // 8, jb, f % 8, pl.ds(c0l, L)] = v
                    return 0

                lax.fori_loop(0, BATCH // L, per_chunk, 0, unroll=False)
                pltpu.sync_copy(tout_v, out_hbm.at[t_base + lt])

    return k


_gather = _make_gather()


B_PER_W = B // NW
CHUNK = 1600
NCHUNK = B_PER_W // CHUNK


def _make_simple_gather():
    mesh = plsc.VectorSubcoreMesh(core_axis_name="c", subcore_axis_name="s")

    @functools.partial(
        pl.kernel,
        mesh=mesh,
        out_type=jax.ShapeDtypeStruct((B, EMBED_DIM), jnp.float32),
        compiler_params=pltpu.CompilerParams(use_tc_tiling_on_sc=False),
        scratch_types=[
            pltpu.VMEM((B_PER_W,), jnp.int32),
            pltpu.VMEM((CHUNK, EMBED_DIM), jnp.float32),
            pltpu.VMEM((CHUNK, EMBED_DIM), jnp.float32),
            pltpu.SemaphoreType.DMA,
            pltpu.SemaphoreType.DMA,
            pltpu.SemaphoreType.DMA,
            pltpu.SemaphoreType.DMA,
        ],
    )
    def k(idx_hbm, table_hbm, out_hbm, idx_v, rows0, rows1, gs0, gs1, ws0, ws1):
        wid = lax.axis_index("s") * NC + lax.axis_index("c")
        base = wid * B_PER_W
        pltpu.sync_copy(idx_hbm.at[pl.ds(base, B_PER_W)], idx_v)

        rows = (rows0, rows1)
        gsem = (gs0, gs1)
        wsem = (ws0, ws1)

        def gather_start(i):
            return pltpu.async_copy(
                table_hbm.at[idx_v.at[pl.ds(i * CHUNK, CHUNK)]],
                rows[i % 2],
                gsem[i % 2],
            )

        def write_start(i):
            return pltpu.async_copy(
                rows[i % 2],
                out_hbm.at[pl.ds(base + i * CHUNK, CHUNK)],
                wsem[i % 2],
            )

        gathers = [gather_start(0), gather_start(1)]
        writes = [None, None]
        for i in range(NCHUNK):
            b = i % 2
            gathers[b].wait()
            writes[b] = write_start(i)
            if i + 2 < NCHUNK:
                writes[b].wait()
                gathers[b] = gather_start(i + 2)
        writes[0].wait()
        writes[1].wait()

    return k


_simple_gather = _make_simple_gather()


@jax.jit
def kernel(x, table):
    tt = table.T
    table_rm = _tc_retile(tt, tt, tt, tt).reshape(_RM_ROWS, EMBED_DIM)
    xt = x.T.reshape(B)
    seg = xt // _SEG
    idx = xt * 4 - seg * (4 * _SEG - 1)
    out = _simple_gather(idx, table_rm)
    return out.reshape(TOKEN_LEN, BATCH, EMBED_DIM).transpose(1, 0, 2)


# speedup vs baseline: 1.8893x; 1.0370x over previous
import functools

import jax
import jax.numpy as jnp
from jax import lax
from jax.experimental import pallas as pl
from jax.experimental.pallas import tpu as pltpu
from jax.experimental.pallas import tpu_sc as plsc

NUM_EMBEDDINGS = 1000000
EMBED_DIM = 32
BATCH = 1024
TOKEN_LEN = 200
B = BATCH * TOKEN_LEN

_info = plsc.get_sparse_core_info()
NC, NS, L = _info.num_cores, _info.num_subcores, _info.num_lanes
NW = NC * NS

_TC_GRID = 62
_TC_W = 4096
_SEG = _TC_GRID * _TC_W
_RM_ROWS = 4 * _SEG


def _tc_retile_body(t0, t1, t2, t3, o_ref):
    o_ref[...] = jnp.concatenate(
        [t0[...].T, t1[...].T, t2[...].T, t3[...].T], axis=1
    )


_tc_retile = pl.pallas_call(
    _tc_retile_body,
    grid=(_TC_GRID,),
    in_specs=[
        pl.BlockSpec(
            (EMBED_DIM, _TC_W),
            lambda g, s=s: (0, jnp.minimum(_TC_GRID * s + g, NUM_EMBEDDINGS // _TC_W)),
        )
        for s in range(4)
    ],
    out_specs=pl.BlockSpec((_TC_W, 128), lambda g: (g, 0)),
    out_shape=jax.ShapeDtypeStruct((_SEG, 128), jnp.float32),
)


def _make_gather():
    mesh = plsc.VectorSubcoreMesh(core_axis_name="c", subcore_axis_name="s")

    @functools.partial(
        pl.kernel,
        mesh=mesh,
        out_type=jax.ShapeDtypeStruct((TOKEN_LEN, 4, 8, 8, 128), jnp.float32),
        compiler_params=pltpu.CompilerParams(
            use_tc_tiling_on_sc=False, needs_layout_passes=False
        ),
        scratch_types=[
            pltpu.VMEM((7 * BATCH,), jnp.int32),
            pltpu.VMEM((BATCH, EMBED_DIM), jnp.float32),
            pltpu.VMEM((BATCH, EMBED_DIM), jnp.float32),
            pltpu.VMEM((4, 8, 8, 128), jnp.float32),
            pltpu.SemaphoreType.DMA,
            pltpu.SemaphoreType.DMA,
        ],
    )
    def k(idx_hbm, table_hbm, out_hbm, idx_v, rows0, rows1, tout_v, gs0, gs1):
        wid = lax.axis_index("s") * NC + lax.axis_index("c")
        nt = jnp.where(wid < 8, 7, 6)
        t_base = jnp.where(wid < 8, wid * 7, 56 + (wid - 8) * 6)

        pltpu.sync_copy(
            idx_hbm.at[pl.ds(t_base * BATCH, 6 * BATCH)],
            idx_v.at[pl.ds(0, 6 * BATCH)],
        )

        @pl.when(nt == 7)
        def _():
            pltpu.sync_copy(
                idx_hbm.at[pl.ds((t_base + 6) * BATCH, BATCH)],
                idx_v.at[pl.ds(6 * BATCH, BATCH)],
            )

        def remap(ci, _):
            v = idx_v[pl.ds(ci * L, L)]
            s = v // _SEG
            idx_v[pl.ds(ci * L, L)] = v * 4 - s * (4 * _SEG - 1)
            return 0

        lax.fori_loop(0, 7 * BATCH // L, remap, 0, unroll=False)

        rows = (rows0, rows1)
        gsem = (gs0, gs1)

        def gather_start(lt):
            return pltpu.async_copy(
                table_hbm.at[idx_v.at[pl.ds(lt * BATCH, BATCH)]],
                rows[lt % 2],
                gsem[lt % 2],
            )

        g = [gather_start(0), None]

        for lt in range(7):

            @pl.when(lt < nt)
            def _(lt=lt):
                if lt + 1 < 7:

                    @pl.when(lt + 1 < nt)
                    def _():
                        g[(lt + 1) % 2] = gather_start(lt + 1)

                g[lt % 2].wait()
                src = rows[lt % 2]

                def per_chunk(cc, _):
                    jb = cc // 8
                    c0l = (cc % 8) * L
                    c_vec = lax.iota(jnp.int32, L) + cc * L
                    for f in range(EMBED_DIM):
                        f_vec = jnp.full((L,), f, jnp.int32)
                        v = plsc.load_gather(src, [c_vec, f_vec])
                        tout_v[f // 8, jb, f % 8, pl.ds(c0l, L)] = v
                    return 0

                lax.fori_loop(0, BATCH // L, per_chunk, 0, unroll=False)
                pltpu.sync_copy(tout_v, out_hbm.at[t_base + lt])

    return k


_gather = _make_gather()


B_PER_W = B // NW
CHUNK = 1600
NCHUNK = B_PER_W // CHUNK


def _make_simple_gather():
    mesh = plsc.VectorSubcoreMesh(core_axis_name="c", subcore_axis_name="s")

    @functools.partial(
        pl.kernel,
        mesh=mesh,
        out_type=jax.ShapeDtypeStruct((B, EMBED_DIM), jnp.float32),
        compiler_params=pltpu.CompilerParams(use_tc_tiling_on_sc=False),
        scratch_types=[
            pltpu.VMEM((B_PER_W,), jnp.int32),
            pltpu.VMEM((CHUNK, EMBED_DIM), jnp.float32),
            pltpu.VMEM((CHUNK, EMBED_DIM), jnp.float32),
            pltpu.SemaphoreType.DMA,
            pltpu.SemaphoreType.DMA,
            pltpu.SemaphoreType.DMA,
            pltpu.SemaphoreType.DMA,
        ],
    )
    def k(idx_hbm, table_hbm, out_hbm, idx_v, rows0, rows1, gs0, gs1, ws0, ws1):
        wid = lax.axis_index("s") * NC + lax.axis_index("c")
        base = wid * B_PER_W
        pltpu.sync_copy(idx_hbm.at[pl.ds(base, B_PER_W)], idx_v)

        rows = (rows0, rows1)
        gsem = (gs0, gs1)
        wsem = (ws0, ws1)

        def gather_start(i):
            return pltpu.async_copy(
                table_hbm.at[idx_v.at[pl.ds(i * CHUNK, CHUNK)]],
                rows[i % 2],
                gsem[i % 2],
            )

        def write_start(i):
            return pltpu.async_copy(
                rows[i % 2],
                out_hbm.at[pl.ds(base + i * CHUNK, CHUNK)],
                wsem[i % 2],
            )

        gathers = [gather_start(0), gather_start(1)]
        writes = [None, None]
        for i in range(NCHUNK):
            b = i % 2
            gathers[b].wait()
            writes[b] = write_start(i)
            if i + 2 < NCHUNK:
                writes[b].wait()
                gathers[b] = gather_start(i + 2)
        writes[0].wait()
        writes[1].wait()

    return k


_simple_gather = _make_simple_gather()


@jax.jit
def kernel(x, table):
    tt = table.T
    table_rm = _tc_retile(tt, tt, tt, tt).reshape(_RM_ROWS, EMBED_DIM)
    xt = x.T.reshape(B)
    seg = xt // _SEG
    idx = xt * 4 - seg * (4 * _SEG - 1)
    out = _simple_gather(idx, table_rm)
    return out.reshape(TOKEN_LEN, BATCH, EMBED_DIM).transpose(1, 0, 2)


# speedup vs baseline: 1.9085x; 1.0101x over previous
import functools

import jax
import jax.numpy as jnp
from jax import lax
from jax.experimental import pallas as pl
from jax.experimental.pallas import tpu as pltpu
from jax.experimental.pallas import tpu_sc as plsc

NUM_EMBEDDINGS = 1000000
EMBED_DIM = 32
BATCH = 1024
TOKEN_LEN = 200
B = BATCH * TOKEN_LEN

_info = plsc.get_sparse_core_info()
NC, NS, L = _info.num_cores, _info.num_subcores, _info.num_lanes
NW = NC * NS

_TC_GRID = 31
_TC_W = 8192
_SEG = _TC_GRID * _TC_W
_RM_ROWS = 4 * _SEG


def _tc_retile_body(t0, t1, t2, t3, o_ref):
    o_ref[...] = jnp.concatenate(
        [t0[...].T, t1[...].T, t2[...].T, t3[...].T], axis=1
    )


_tc_retile = pl.pallas_call(
    _tc_retile_body,
    grid=(_TC_GRID,),
    in_specs=[
        pl.BlockSpec(
            (EMBED_DIM, _TC_W),
            lambda g, s=s: (0, jnp.minimum(_TC_GRID * s + g, NUM_EMBEDDINGS // _TC_W)),
        )
        for s in range(4)
    ],
    out_specs=pl.BlockSpec((_TC_W, 128), lambda g: (g, 0)),
    out_shape=jax.ShapeDtypeStruct((_SEG, 128), jnp.float32),
)


def _make_gather():
    mesh = plsc.VectorSubcoreMesh(core_axis_name="c", subcore_axis_name="s")

    @functools.partial(
        pl.kernel,
        mesh=mesh,
        out_type=jax.ShapeDtypeStruct((TOKEN_LEN, 4, 8, 8, 128), jnp.float32),
        compiler_params=pltpu.CompilerParams(
            use_tc_tiling_on_sc=False, needs_layout_passes=False
        ),
        scratch_types=[
            pltpu.VMEM((7 * BATCH,), jnp.int32),
            pltpu.VMEM((BATCH, EMBED_DIM), jnp.float32),
            pltpu.VMEM((BATCH, EMBED_DIM), jnp.float32),
            pltpu.VMEM((4, 8, 8, 128), jnp.float32),
            pltpu.SemaphoreType.DMA,
            pltpu.SemaphoreType.DMA,
        ],
    )
    def k(idx_hbm, table_hbm, out_hbm, idx_v, rows0, rows1, tout_v, gs0, gs1):
        wid = lax.axis_index("s") * NC + lax.axis_index("c")
        nt = jnp.where(wid < 8, 7, 6)
        t_base = jnp.where(wid < 8, wid * 7, 56 + (wid - 8) * 6)

        pltpu.sync_copy(
            idx_hbm.at[pl.ds(t_base * BATCH, 6 * BATCH)],
            idx_v.at[pl.ds(0, 6 * BATCH)],
        )

        @pl.when(nt == 7)
        def _():
            pltpu.sync_copy(
                idx_hbm.at[pl.ds((t_base + 6) * BATCH, BATCH)],
                idx_v.at[pl.ds(6 * BATCH, BATCH)],
            )

        def remap(ci, _):
            v = idx_v[pl.ds(ci * L, L)]
            s = v // _SEG
            idx_v[pl.ds(ci * L, L)] = v * 4 - s * (4 * _SEG - 1)
            return 0

        lax.fori_loop(0, 7 * BATCH // L, remap, 0, unroll=False)

        rows = (rows0, rows1)
        gsem = (gs0, gs1)

        def gather_start(lt):
            return pltpu.async_copy(
                table_hbm.at[idx_v.at[pl.ds(lt * BATCH, BATCH)]],
                rows[lt % 2],
                gsem[lt % 2],
            )

        g = [gather_start(0), None]

        for lt in range(7):

            @pl.when(lt < nt)
            def _(lt=lt):
                if lt + 1 < 7:

                    @pl.when(lt + 1 < nt)
                    def _():
                        g[(lt + 1) % 2] = gather_start(lt + 1)

                g[lt % 2].wait()
                src = rows[lt % 2]

                def per_chunk(cc, _):
                    jb = cc // 8
                    c0l = (cc % 8) * L
                    c_vec = lax.iota(jnp.int32, L) + cc * L
                    for f in range(EMBED_DIM):
                        f_vec = jnp.full((L,), f, jnp.int32)
                        v = plsc.load_gather(src, [c_vec, f_vec])
                        tout_v[f // 8, jb, f % 8, pl.ds(c0l, L)] = v
                    return 0

                lax.fori_loop(0, BATCH // L, per_chunk, 0, unroll=False)
                pltpu.sync_copy(tout_v, out_hbm.at[t_base + lt])

    return k


_gather = _make_gather()


B_PER_W = B // NW
CHUNK = 1600
NCHUNK = B_PER_W // CHUNK


def _make_simple_gather():
    mesh = plsc.VectorSubcoreMesh(core_axis_name="c", subcore_axis_name="s")

    @functools.partial(
        pl.kernel,
        mesh=mesh,
        out_type=jax.ShapeDtypeStruct((B, EMBED_DIM), jnp.float32),
        compiler_params=pltpu.CompilerParams(use_tc_tiling_on_sc=False),
        scratch_types=[
            pltpu.VMEM((B_PER_W,), jnp.int32),
            pltpu.VMEM((CHUNK, EMBED_DIM), jnp.float32),
            pltpu.VMEM((CHUNK, EMBED_DIM), jnp.float32),
            pltpu.SemaphoreType.DMA,
            pltpu.SemaphoreType.DMA,
            pltpu.SemaphoreType.DMA,
            pltpu.SemaphoreType.DMA,
        ],
    )
    def k(idx_hbm, table_hbm, out_hbm, idx_v, rows0, rows1, gs0, gs1, ws0, ws1):
        wid = lax.axis_index("s") * NC + lax.axis_index("c")
        base = wid * B_PER_W
        pltpu.sync_copy(idx_hbm.at[pl.ds(base, B_PER_W)], idx_v)

        rows = (rows0, rows1)
        gsem = (gs0, gs1)
        wsem = (ws0, ws1)

        def gather_start(i):
            return pltpu.async_copy(
                table_hbm.at[idx_v.at[pl.ds(i * CHUNK, CHUNK)]],
                rows[i % 2],
                gsem[i % 2],
            )

        def write_start(i):
            return pltpu.async_copy(
                rows[i % 2],
                out_hbm.at[pl.ds(base + i * CHUNK, CHUNK)],
                wsem[i % 2],
            )

        gathers = [gather_start(0), gather_start(1)]
        writes = [None, None]
        for i in range(NCHUNK):
            b = i % 2
            gathers[b].wait()
            writes[b] = write_start(i)
            if i + 2 < NCHUNK:
                writes[b].wait()
                gathers[b] = gather_start(i + 2)
        writes[0].wait()
        writes[1].wait()

    return k


_simple_gather = _make_simple_gather()


@jax.jit
def kernel(x, table):
    tt = table.T
    table_rm = _tc_retile(tt, tt, tt, tt).reshape(_RM_ROWS, EMBED_DIM)
    xt = x.T.reshape(B)
    seg = xt // _SEG
    idx = xt * 4 - seg * (4 * _SEG - 1)
    out = _simple_gather(idx, table_rm)
    return out.reshape(TOKEN_LEN, BATCH, EMBED_DIM).transpose(1, 0, 2)
